# Initial kernel scaffold; baseline (speedup 1.0000x reference)
import jax
import jax.numpy as jnp
from jax.experimental import pallas as pl


def kernel(speaker_labels, table):
    raise NotImplementedError("write your pallas kernel here")


# speedup vs baseline: 5.4061x; 5.4061x over previous
import functools

import jax
import jax.numpy as jnp
from jax import lax
from jax.experimental import pallas as pl
from jax.experimental.pallas import tpu as pltpu
from jax.experimental.pallas import tpu_sc as plsc

R, C = 16384, 200
D = 32
N = R * C
NW = 32
B = 1024
IPW = N // NW
NCH = IPW // B
L = 16


def _sc_lookup(labels_2d, table_flat):
    mesh = plsc.VectorSubcoreMesh(core_axis_name="c", subcore_axis_name="s")

    @functools.partial(
        pl.kernel,
        mesh=mesh,
        out_type=jax.ShapeDtypeStruct((N // B, B * D), jnp.float32),
        scratch_types=[
            pltpu.VMEM((128,), jnp.float32),
            pltpu.VMEM((B,), jnp.int32),
            pltpu.VMEM((B * D,), jnp.float32),
        ],
    )
    def k(labels_hbm, table_hbm, out_hbm, tab_v, idx_v, rows_v):
        wid = lax.axis_index("s") * 2 + lax.axis_index("c")
        base = wid * NCH
        pltpu.sync_copy(table_hbm, tab_v)

        def chunk(c, carry):
            g = base + c
            pltpu.sync_copy(labels_hbm.at[g], idx_v)

            def body(i, carry2):
                iv = idx_v[pl.ds(i * L, L)]
                offs = iv * D
                for u in range(L):
                    o = offs[u]
                    r = (i * L + u) * D
                    rows_v[pl.ds(r, L)] = tab_v[pl.ds(o, L)]
                    rows_v[pl.ds(r + L, L)] = tab_v[pl.ds(o + L, L)]
                return carry2

            lax.fori_loop(0, B // L, body, 0)
            pltpu.sync_copy(rows_v, out_hbm.at[g])
            return carry

        lax.fori_loop(0, NCH, chunk, 0)

    return k(labels_2d, table_flat)


def kernel(speaker_labels, table):
    t = jnp.zeros((128,), jnp.float32).at[: 3 * D].set(
        table.at[0].set(0.0).reshape(-1)
    )
    labels = speaker_labels.astype(jnp.int32).reshape(N // B, B)
    out = _sc_lookup(labels, t)
    return out.reshape(R, C, D)
